# baseline (speedup 1.0000x reference)
import functools

import jax
import jax.numpy as jnp
from jax import lax
from jax.experimental import pallas as pl
from jax.experimental.pallas import tpu as pltpu
from jax.experimental.pallas import tpu_sc as plsc

N_NODES = 100000
N_EDGES = 3200000
IN_FEATS = 128
ADDED = 24
OUT_FEATS = 128

FG = 8
N_GROUPS = ADDED // FG
LANES = 128
IDX_ROWS = N_EDGES // LANES
CHUNK_ROWS = 2
CHUNK_E = CHUNK_ROWS * LANES
N_CHUNKS = IDX_ROWS // CHUNK_ROWS
HALF_CHUNKS = N_CHUNKS // 2
N_TILES = 16
ROWS_PER_TILE = N_NODES // N_TILES
NBUF = 3
NPIECE = 8
N_SLOTS = 786
N_SUPER = N_SLOTS // 6


def _sc_segment_sum(em4d, dst2d, zeros):
    mesh = plsc.VectorSubcoreMesh(core_axis_name="c", subcore_axis_name="s")

    @functools.partial(
        pl.kernel,
        out_type=tuple(jax.ShapeDtypeStruct((N_NODES, FG), jnp.float32)
                       for _ in range(4)),
        mesh=mesh,
        scratch_types=[
            pltpu.VMEM((NBUF, CHUNK_ROWS, LANES), jnp.int32),
            pltpu.VMEM((NBUF, 2, CHUNK_ROWS, FG, LANES), jnp.float32),
            pltpu.VMEM((NPIECE, LANES, FG), jnp.float32),
            pltpu.VMEM_SHARED((N_NODES, FG), jnp.float32),
            pltpu.VMEM_SHARED((N_NODES, FG), jnp.float32),
        ] + [pltpu.SemaphoreType.DMA] * (NBUF + NPIECE),
        compiler_params=pltpu.CompilerParams(use_tc_tiling_on_sc=False,
                                             needs_layout_passes=False),
    )
    def run(em_hbm, dst_hbm, zero_hbm, out0_hbm, out1a_hbm, out1b_hbm,
            out2_hbm, idx_v, slab_v, rows_v, accp, accs, *sems):
        c = lax.axis_index("c")
        t = lax.axis_index("s")
        node0 = t * ROWS_PER_TILE
        lsem = sems[:NBUF]
        ssem = sems[NBUF:]

        pltpu.sync_copy(zero_hbm, accp.at[pl.ds(node0, ROWS_PER_TILE), :])
        pltpu.sync_copy(zero_hbm, accs.at[pl.ds(node0, ROWS_PER_TILE), :])
        plsc.subcore_barrier()

        ii = lax.broadcasted_iota(jnp.int32, (16,), 0)
        colv = [jnp.full((16,), f, jnp.int32) for f in range(FG)]

        def chunk_of(n):
            return t + n * N_TILES

        def valid(n):
            return chunk_of(n) < N_CHUNKS

        def sec(n):
            ch = chunk_of(n)
            return jnp.where(c == 0, ch < HALF_CHUNKS,
                             (ch >= HALF_CHUNKS) & (ch < N_CHUNKS))

        def load_descs(b, n):
            ch = chunk_of(n)
            row0 = ch * CHUNK_ROWS
            idx_d = (dst_hbm.at[pl.ds(row0, CHUNK_ROWS), :], idx_v.at[b],
                     lsem[b])
            p0_d = (em_hbm.at[0, pl.ds(row0, CHUNK_ROWS)],
                    slab_v.at[b, 0], lsem[b])
            p2_d = (em_hbm.at[2, pl.ds(row0, CHUNK_ROWS)],
                    slab_v.at[b, 0], lsem[b])
            s_d = (em_hbm.at[1, pl.ds(row0, CHUNK_ROWS)],
                   slab_v.at[b, 1], lsem[b])
            return idx_d, p0_d, p2_d, s_d

        def start_loads(b, n):
            idx_d, p0_d, p2_d, s_d = load_descs(b, n)
            pltpu.async_copy(*idx_d)

            @pl.when(c == 0)
            def _():
                pltpu.async_copy(*p0_d)

            @pl.when(c == 1)
            def _():
                pltpu.async_copy(*p2_d)

            @pl.when(sec(n))
            def _():
                pltpu.async_copy(*s_d)

        def wait_loads(b, n):
            idx_d, p0_d, _, s_d = load_descs(b, n)
            pltpu.make_async_copy(*idx_d).wait()
            pltpu.make_async_copy(*p0_d).wait()

            @pl.when(sec(n))
            def _():
                pltpu.make_async_copy(*s_d).wait()

        def sct_desc(p, b, r, grp):
            acc = accp if grp == 0 else accs
            return (rows_v.at[p], acc.at[idx_v.at[b, r]], ssem[p])

        def transpose_piece(b, grp, r, p):
            slab_r = slab_v.at[b, grp, r]

            def tr_body(h, carry):
                rv = ii + h * 16
                base = h * 16
                for f in range(FG):
                    v = slab_r[f, pl.ds(base, 16)]
                    plsc.store_scatter(rows_v.at[p], [rv, colv[f]], v)
                return carry

            lax.fori_loop(0, LANES // 16, tr_body, 0, unroll=4)

        def slot(n, b, q):
            qb = 4 * (1 - q)
            for r in range(CHUNK_ROWS):
                @pl.when((n >= 1) & valid(n - 1))
                def _(r=r):
                    pltpu.make_async_copy(*sct_desc(qb + r, 0, 0, 0)).wait()

                @pl.when((n >= 1) & valid(n - 1) & sec(n - 1))
                def _(r=r):
                    pltpu.make_async_copy(
                        *sct_desc(qb + 2 + r, 0, 0, 0)).wait()

            @pl.when(valid(n))
            def _():
                wait_loads(b, n)
                for r in range(CHUNK_ROWS):
                    p = 4 * q + r
                    transpose_piece(b, 0, r, p)
                    pltpu.async_copy(*sct_desc(p, b, r, 0), add=True)

            @pl.when(valid(n) & sec(n))
            def _():
                for r in range(CHUNK_ROWS):
                    p = 4 * q + 2 + r
                    transpose_piece(b, 1, r, p)
                    pltpu.async_copy(*sct_desc(p, b, r, 1), add=True)

            @pl.when(valid(n + 2))
            def _():
                start_loads((b + 2) % NBUF, n + 2)

        for n0 in range(2):
            @pl.when(valid(n0))
            def _(n0=n0):
                start_loads(n0, n0)

        def superstep(s, carry):
            for k in range(6):
                slot(s * 6 + k, k % 3, k % 2)
            return carry

        lax.fori_loop(0, N_SUPER, superstep, 0)

        plsc.subcore_barrier()

        nslice = pl.ds(node0, ROWS_PER_TILE)

        @pl.when(c == 0)
        def _():
            pltpu.sync_copy(accp.at[nslice, :], out0_hbm.at[nslice, :])
            pltpu.sync_copy(accs.at[nslice, :], out1a_hbm.at[nslice, :])

        @pl.when(c == 1)
        def _():
            pltpu.sync_copy(accp.at[nslice, :], out2_hbm.at[nslice, :])
            pltpu.sync_copy(accs.at[nslice, :], out1b_hbm.at[nslice, :])

    return run(em4d, dst2d, zeros)


def _tc_dense(h, g0, g1a, g1b, g2, norm, w1t, w20t, w21t, w22t, b2, gg2,
              be2):
    BR = 1000
    grid = N_NODES // BR

    def body(h_ref, g0_ref, g1a_ref, g1b_ref, g2_ref, n_ref, w1_ref,
             w20_ref, w21_ref, w22_ref, b_ref, g_ref, be_ref, o_ref):
        nb = n_ref[...]
        x = (jnp.dot(h_ref[...], w1_ref[...],
                     preferred_element_type=jnp.float32)
             + jnp.dot(g0_ref[...] * nb, w20_ref[...],
                       preferred_element_type=jnp.float32)
             + jnp.dot((g1a_ref[...] + g1b_ref[...]) * nb, w21_ref[...],
                       preferred_element_type=jnp.float32)
             + jnp.dot(g2_ref[...] * nb, w22_ref[...],
                       preferred_element_type=jnp.float32)
             + b_ref[...])
        mu = jnp.mean(x, axis=1, keepdims=True)
        xc = x - mu
        var = jnp.mean(xc * xc, axis=1, keepdims=True)
        y = xc * lax.rsqrt(var + 1e-5) * g_ref[...] + be_ref[...]
        o_ref[...] = jnp.maximum(y, 0.0)

    gspec = pl.BlockSpec((BR, FG), lambda i: (i, 0))
    wspec = pl.BlockSpec((FG, OUT_FEATS), lambda i: (0, 0))
    vspec = pl.BlockSpec((1, OUT_FEATS), lambda i: (0, 0))
    return pl.pallas_call(
        body,
        grid=(grid,),
        in_specs=[
            pl.BlockSpec((BR, IN_FEATS), lambda i: (i, 0)),
            gspec, gspec, gspec, gspec,
            pl.BlockSpec((BR, 1), lambda i: (i, 0)),
            pl.BlockSpec((IN_FEATS, OUT_FEATS), lambda i: (0, 0)),
            wspec, wspec, wspec,
            vspec, vspec, vspec,
        ],
        out_specs=pl.BlockSpec((BR, OUT_FEATS), lambda i: (i, 0)),
        out_shape=jax.ShapeDtypeStruct((N_NODES, OUT_FEATS), jnp.float32),
    )(h, g0, g1a, g1b, g2, norm, w1t, w20t, w21t, w22t, b2, gg2, be2)


def kernel(h, edge_m, norm, edge_index, W, b, ln_g, ln_b):
    dst2d = edge_index[1].astype(jnp.int32).reshape(IDX_ROWS, LANES)
    em4d = edge_m.T.reshape(N_GROUPS, FG, IDX_ROWS, LANES).transpose(
        0, 2, 1, 3)
    zeros = jnp.zeros((ROWS_PER_TILE, FG), jnp.float32)
    g0, g1a, g1b, g2 = _sc_segment_sum(em4d, dst2d, zeros)
    w1t = W[:, :IN_FEATS].T
    w20t = W[:, IN_FEATS:IN_FEATS + FG].T
    w21t = W[:, IN_FEATS + FG:IN_FEATS + 2 * FG].T
    w22t = W[:, IN_FEATS + 2 * FG:].T
    return _tc_dense(h, g0, g1a, g1b, g2, norm, w1t, w20t, w21t, w22t,
                     b.reshape(1, -1), ln_g.reshape(1, -1),
                     ln_b.reshape(1, -1))


# speedup vs baseline: 1.0342x; 1.0342x over previous
import functools

import jax
import jax.numpy as jnp
from jax import lax
from jax.experimental import pallas as pl
from jax.experimental.pallas import tpu as pltpu
from jax.experimental.pallas import tpu_sc as plsc

N_NODES = 100000
N_EDGES = 3200000
IN_FEATS = 128
ADDED = 24
OUT_FEATS = 128

FG = 8
N_GROUPS = ADDED // FG
LANES = 128
IDX_ROWS = N_EDGES // LANES
CHUNK_ROWS = 2
CHUNK_E = CHUNK_ROWS * LANES
N_CHUNKS = IDX_ROWS // CHUNK_ROWS
HALF_CHUNKS = N_CHUNKS // 2
N_TILES = 16
ROWS_PER_TILE = N_NODES // N_TILES
NBUF = 3
NPIECE = 8
N_SLOTS = 786
N_SUPER = N_SLOTS // 6


def _sc_segment_sum(em4d, dst2d, zeros):
    mesh = plsc.VectorSubcoreMesh(core_axis_name="c", subcore_axis_name="s")

    @functools.partial(
        pl.kernel,
        out_type=tuple(jax.ShapeDtypeStruct((N_NODES, FG), jnp.float32)
                       for _ in range(4)),
        mesh=mesh,
        scratch_types=[
            pltpu.VMEM((NBUF, CHUNK_ROWS, LANES), jnp.int32),
            pltpu.VMEM((NBUF, 2, CHUNK_ROWS, FG, LANES), jnp.float32),
            pltpu.VMEM((NPIECE, LANES, FG), jnp.float32),
            pltpu.VMEM_SHARED((N_NODES, FG), jnp.float32),
            pltpu.VMEM_SHARED((N_NODES, FG), jnp.float32),
        ] + [pltpu.SemaphoreType.DMA] * (NBUF + NPIECE),
        compiler_params=pltpu.CompilerParams(use_tc_tiling_on_sc=False,
                                             needs_layout_passes=False),
    )
    def run(em_hbm, dst_hbm, zero_hbm, out0_hbm, out1a_hbm, out1b_hbm,
            out2_hbm, idx_v, slab_v, rows_v, accp, accs, *sems):
        c = lax.axis_index("c")
        t = lax.axis_index("s")
        node0 = t * ROWS_PER_TILE
        lsem = sems[:NBUF]
        ssem = sems[NBUF:]

        pltpu.sync_copy(zero_hbm, accp.at[pl.ds(node0, ROWS_PER_TILE), :])
        pltpu.sync_copy(zero_hbm, accs.at[pl.ds(node0, ROWS_PER_TILE), :])
        plsc.subcore_barrier()

        ii = lax.broadcasted_iota(jnp.int32, (16,), 0)
        colv = [jnp.full((16,), f, jnp.int32) for f in range(FG)]

        def chunk_of(n):
            return t + n * N_TILES

        def valid(n):
            return chunk_of(n) < N_CHUNKS

        def sec(n):
            ch = chunk_of(n)
            return jnp.where(c == 0, ch < HALF_CHUNKS,
                             (ch >= HALF_CHUNKS) & (ch < N_CHUNKS))

        def load_descs(b, n):
            ch = chunk_of(n)
            row0 = ch * CHUNK_ROWS
            idx_d = (dst_hbm.at[pl.ds(row0, CHUNK_ROWS), :], idx_v.at[b],
                     lsem[b])
            p0_d = (em_hbm.at[0, pl.ds(row0, CHUNK_ROWS)],
                    slab_v.at[b, 0], lsem[b])
            p2_d = (em_hbm.at[2, pl.ds(row0, CHUNK_ROWS)],
                    slab_v.at[b, 0], lsem[b])
            s_d = (em_hbm.at[1, pl.ds(row0, CHUNK_ROWS)],
                   slab_v.at[b, 1], lsem[b])
            return idx_d, p0_d, p2_d, s_d

        def start_loads(b, n):
            idx_d, p0_d, p2_d, s_d = load_descs(b, n)
            pltpu.async_copy(*idx_d)

            @pl.when(c == 0)
            def _():
                pltpu.async_copy(*p0_d)

            @pl.when(c == 1)
            def _():
                pltpu.async_copy(*p2_d)

            @pl.when(sec(n))
            def _():
                pltpu.async_copy(*s_d)

        def wait_loads(b, n):
            idx_d, p0_d, _, s_d = load_descs(b, n)
            pltpu.make_async_copy(*idx_d).wait()
            pltpu.make_async_copy(*p0_d).wait()

            @pl.when(sec(n))
            def _():
                pltpu.make_async_copy(*s_d).wait()

        def sct_desc(p, b, r, grp):
            acc = accp if grp == 0 else accs
            return (rows_v.at[p], acc.at[idx_v.at[b, r]], ssem[p])

        def transpose_piece(b, grp, r, p):
            slab_r = slab_v.at[b, grp, r]

            def tr_body(h, carry):
                rv = ii + h * 16
                base = h * 16
                for f in range(FG):
                    v = slab_r[f, pl.ds(base, 16)]
                    plsc.store_scatter(rows_v.at[p], [rv, colv[f]], v)
                return carry

            lax.fori_loop(0, LANES // 16, tr_body, 0)

        def slot(n, b, q):
            qb = 4 * (1 - q)
            for r in range(CHUNK_ROWS):
                @pl.when((n >= 1) & valid(n - 1))
                def _(r=r):
                    pltpu.make_async_copy(*sct_desc(qb + r, 0, 0, 0)).wait()

                @pl.when((n >= 1) & valid(n - 1) & sec(n - 1))
                def _(r=r):
                    pltpu.make_async_copy(
                        *sct_desc(qb + 2 + r, 0, 0, 0)).wait()

            @pl.when(valid(n))
            def _():
                wait_loads(b, n)
                for r in range(CHUNK_ROWS):
                    p = 4 * q + r
                    transpose_piece(b, 0, r, p)
                    pltpu.async_copy(*sct_desc(p, b, r, 0), add=True)

            @pl.when(valid(n) & sec(n))
            def _():
                for r in range(CHUNK_ROWS):
                    p = 4 * q + 2 + r
                    transpose_piece(b, 1, r, p)
                    pltpu.async_copy(*sct_desc(p, b, r, 1), add=True)

            @pl.when(valid(n + 2))
            def _():
                start_loads((b + 2) % NBUF, n + 2)

        for n0 in range(2):
            @pl.when(valid(n0))
            def _(n0=n0):
                start_loads(n0, n0)

        def superstep(s, carry):
            for k in range(6):
                slot(s * 6 + k, k % 3, k % 2)
            return carry

        lax.fori_loop(0, N_SUPER, superstep, 0)

        plsc.subcore_barrier()

        nslice = pl.ds(node0, ROWS_PER_TILE)

        @pl.when(c == 0)
        def _():
            pltpu.sync_copy(accp.at[nslice, :], out0_hbm.at[nslice, :])
            pltpu.sync_copy(accs.at[nslice, :], out1a_hbm.at[nslice, :])

        @pl.when(c == 1)
        def _():
            pltpu.sync_copy(accp.at[nslice, :], out2_hbm.at[nslice, :])
            pltpu.sync_copy(accs.at[nslice, :], out1b_hbm.at[nslice, :])

    return run(em4d, dst2d, zeros)


def _tc_dense1(h, w1t, b2):
    BR = 1000
    grid = N_NODES // BR

    def body(h_ref, w1_ref, b_ref, o_ref):
        o_ref[...] = jnp.dot(h_ref[...], w1_ref[...],
                             preferred_element_type=jnp.float32) + b_ref[...]

    return pl.pallas_call(
        body,
        grid=(grid,),
        in_specs=[
            pl.BlockSpec((BR, IN_FEATS), lambda i: (i, 0)),
            pl.BlockSpec((IN_FEATS, OUT_FEATS), lambda i: (0, 0)),
            pl.BlockSpec((1, OUT_FEATS), lambda i: (0, 0)),
        ],
        out_specs=pl.BlockSpec((BR, OUT_FEATS), lambda i: (i, 0)),
        out_shape=jax.ShapeDtypeStruct((N_NODES, OUT_FEATS), jnp.float32),
    )(h, w1t, b2)


def _tc_dense2(y1, g0, g1a, g1b, g2, norm, w20t, w21t, w22t, gg2, be2):
    BR = 1000
    grid = N_NODES // BR

    def body(y1_ref, g0_ref, g1a_ref, g1b_ref, g2_ref, n_ref,
             w20_ref, w21_ref, w22_ref, g_ref, be_ref, o_ref):
        nb = n_ref[...]
        x = (y1_ref[...]
             + jnp.dot(g0_ref[...] * nb, w20_ref[...],
                       preferred_element_type=jnp.float32)
             + jnp.dot((g1a_ref[...] + g1b_ref[...]) * nb, w21_ref[...],
                       preferred_element_type=jnp.float32)
             + jnp.dot(g2_ref[...] * nb, w22_ref[...],
                       preferred_element_type=jnp.float32))
        mu = jnp.mean(x, axis=1, keepdims=True)
        xc = x - mu
        var = jnp.mean(xc * xc, axis=1, keepdims=True)
        y = xc * lax.rsqrt(var + 1e-5) * g_ref[...] + be_ref[...]
        o_ref[...] = jnp.maximum(y, 0.0)

    gspec = pl.BlockSpec((BR, FG), lambda i: (i, 0))
    wspec = pl.BlockSpec((FG, OUT_FEATS), lambda i: (0, 0))
    vspec = pl.BlockSpec((1, OUT_FEATS), lambda i: (0, 0))
    return pl.pallas_call(
        body,
        grid=(grid,),
        in_specs=[
            pl.BlockSpec((BR, OUT_FEATS), lambda i: (i, 0)),
            gspec, gspec, gspec, gspec,
            pl.BlockSpec((BR, 1), lambda i: (i, 0)),
            wspec, wspec, wspec,
            vspec, vspec,
        ],
        out_specs=pl.BlockSpec((BR, OUT_FEATS), lambda i: (i, 0)),
        out_shape=jax.ShapeDtypeStruct((N_NODES, OUT_FEATS), jnp.float32),
    )(y1, g0, g1a, g1b, g2, norm, w20t, w21t, w22t, gg2, be2)


def kernel(h, edge_m, norm, edge_index, W, b, ln_g, ln_b):
    dst2d = edge_index[1].astype(jnp.int32).reshape(IDX_ROWS, LANES)
    em4d = edge_m.T.reshape(N_GROUPS, FG, IDX_ROWS, LANES).transpose(
        0, 2, 1, 3)
    zeros = jnp.zeros((ROWS_PER_TILE, FG), jnp.float32)
    g0, g1a, g1b, g2 = _sc_segment_sum(em4d, dst2d, zeros)
    w1t = W[:, :IN_FEATS].T
    w20t = W[:, IN_FEATS:IN_FEATS + FG].T
    w21t = W[:, IN_FEATS + FG:IN_FEATS + 2 * FG].T
    w22t = W[:, IN_FEATS + 2 * FG:].T
    y1 = _tc_dense1(h, w1t, b.reshape(1, -1))
    return _tc_dense2(y1, g0, g1a, g1b, g2, norm, w20t, w21t, w22t,
                      ln_g.reshape(1, -1), ln_b.reshape(1, -1))
